# baseline (speedup 1.0000x reference)
import functools

import jax
import jax.numpy as jnp
from jax.experimental import pallas as pl
from jax.experimental.pallas import tpu as pltpu

_EPS = 1e-5


def _typenorm_body(t_ref, x_ref, g_ref, b_ref, o_ref, *, num_types, chunk):
    gam = g_ref[...]
    bet = b_ref[...]
    n_chunks = x_ref.shape[0] // chunk

    def body(c, carry):
        sl = pl.ds(c * chunk, chunk)
        x = x_ref[sl, :]
        mean = jnp.mean(x, axis=1, keepdims=True)
        xc = x - mean
        var = jnp.mean(xc * xc, axis=1, keepdims=True)
        xhat = xc * jax.lax.rsqrt(var + _EPS)
        t = t_ref[sl, :]
        onehot = (t == jax.lax.broadcasted_iota(
            jnp.int32, (chunk, num_types), 1)).astype(jnp.float32)
        g = jnp.dot(onehot, gam, preferred_element_type=jnp.float32)
        b = jnp.dot(onehot, bet, preferred_element_type=jnp.float32)
        o_ref[sl, :] = xhat * g + b
        return carry

    jax.lax.fori_loop(0, n_chunks, body, 0)


def kernel(type_list, abstract_features, gamma, beta):
    n, d = abstract_features.shape
    num_types = gamma.shape[0]
    t2 = type_list.astype(jnp.int32).reshape(n, 1)

    block_n = 16000
    chunk = 2000
    if n % block_n != 0:
        block_n = 1024
        chunk = 1024
    grid = (pl.cdiv(n, block_n),)

    return pl.pallas_call(
        functools.partial(_typenorm_body, num_types=num_types, chunk=chunk),
        out_shape=jax.ShapeDtypeStruct((n, d), jnp.float32),
        grid=grid,
        in_specs=[
            pl.BlockSpec((block_n, 1), lambda i: (i, 0)),
            pl.BlockSpec((block_n, d), lambda i: (i, 0)),
            pl.BlockSpec((num_types, d), lambda i: (0, 0)),
            pl.BlockSpec((num_types, d), lambda i: (0, 0)),
        ],
        out_specs=pl.BlockSpec((block_n, d), lambda i: (i, 0)),
        compiler_params=pltpu.CompilerParams(
            dimension_semantics=("parallel",),
        ),
        name="typenorm",
    )(t2, abstract_features, gamma, beta)


# speedup vs baseline: 1.0851x; 1.0851x over previous
import functools

import jax
import jax.numpy as jnp
from jax.experimental import pallas as pl
from jax.experimental.pallas import tpu as pltpu

_EPS = 1e-5
_NBUF = 4
_PREFETCH = 2


def _compute(x, t, gam, bet, num_types):
    mean = jnp.mean(x, axis=1, keepdims=True)
    xc = x - mean
    var = jnp.mean(xc * xc, axis=1, keepdims=True)
    xhat = xc * jax.lax.rsqrt(var + _EPS)
    onehot = (t == jax.lax.broadcasted_iota(
        jnp.int32, (t.shape[0], num_types), 1)).astype(jnp.float32)
    g = jnp.dot(onehot, gam, preferred_element_type=jnp.float32)
    b = jnp.dot(onehot, bet, preferred_element_type=jnp.float32)
    return xhat * g + b


def _ring_body(t_hbm, x_hbm, g_ref, b_ref, o_hbm,
               x_buf, t_buf, o_buf, in_sem, t_sem, out_sem,
               *, num_types, block, steps):
    gam = g_ref[...]
    bet = b_ref[...]

    def start_in(slot, step):
        pltpu.make_async_copy(
            x_hbm.at[pl.ds(step * block, block)], x_buf.at[slot],
            in_sem.at[slot]).start()
        pltpu.make_async_copy(
            t_hbm.at[pl.ds(step * block, block)], t_buf.at[slot],
            t_sem.at[slot]).start()

    def wait_in(slot):
        pltpu.make_async_copy(
            x_hbm.at[pl.ds(0, block)], x_buf.at[slot],
            in_sem.at[slot]).wait()
        pltpu.make_async_copy(
            t_hbm.at[pl.ds(0, block)], t_buf.at[slot],
            t_sem.at[slot]).wait()

    def start_out(slot, step):
        pltpu.make_async_copy(
            o_buf.at[slot], o_hbm.at[pl.ds(step * block, block)],
            out_sem.at[slot]).start()

    def wait_out(slot):
        pltpu.make_async_copy(
            o_buf.at[slot], o_hbm.at[pl.ds(0, block)],
            out_sem.at[slot]).wait()

    for s in range(_PREFETCH):
        start_in(s % _NBUF, s)

    def body(step, carry):
        slot = jax.lax.rem(step, _NBUF)

        @pl.when(step >= _NBUF)
        def _():
            wait_out(slot)

        @pl.when(step + _PREFETCH < steps)
        def _():
            start_in(jax.lax.rem(step + _PREFETCH, _NBUF), step + _PREFETCH)

        wait_in(slot)
        x = x_buf[slot]
        t = t_buf[slot]
        o_buf[slot] = _compute(x, t, gam, bet, num_types)
        start_out(slot, step)
        return carry

    jax.lax.fori_loop(0, steps, body, 0)

    for s in range(max(steps - _NBUF, 0), steps):
        wait_out(s % _NBUF)


def kernel(type_list, abstract_features, gamma, beta):
    n, d = abstract_features.shape
    num_types = gamma.shape[0]
    t2 = type_list.astype(jnp.int32).reshape(n, 1)

    block = 4000
    if n % block != 0:
        block = 1024
    steps = n // block

    return pl.pallas_call(
        functools.partial(_ring_body, num_types=num_types, block=block,
                          steps=steps),
        out_shape=jax.ShapeDtypeStruct((n, d), jnp.float32),
        in_specs=[
            pl.BlockSpec(memory_space=pltpu.MemorySpace.HBM),
            pl.BlockSpec(memory_space=pltpu.MemorySpace.HBM),
            pl.BlockSpec(memory_space=pltpu.VMEM),
            pl.BlockSpec(memory_space=pltpu.VMEM),
        ],
        out_specs=pl.BlockSpec(memory_space=pltpu.MemorySpace.HBM),
        scratch_shapes=[
            pltpu.VMEM((_NBUF, block, d), jnp.float32),
            pltpu.VMEM((_NBUF, block, 1), jnp.int32),
            pltpu.VMEM((_NBUF, block, d), jnp.float32),
            pltpu.SemaphoreType.DMA((_NBUF,)),
            pltpu.SemaphoreType.DMA((_NBUF,)),
            pltpu.SemaphoreType.DMA((_NBUF,)),
        ],
        name="typenorm_ring",
    )(t2, abstract_features, gamma, beta)


# speedup vs baseline: 1.0898x; 1.0043x over previous
import functools

import jax
import jax.numpy as jnp
from jax.experimental import pallas as pl
from jax.experimental.pallas import tpu as pltpu

_EPS = 1e-5
_NBUF = 6
_PREFETCH = 3


def _compute(x, t, gam, bet, num_types):
    mean = jnp.mean(x, axis=1, keepdims=True)
    xc = x - mean
    var = jnp.mean(xc * xc, axis=1, keepdims=True)
    xhat = xc * jax.lax.rsqrt(var + _EPS)
    onehot = (t == jax.lax.broadcasted_iota(
        jnp.int32, (t.shape[0], num_types), 1)).astype(jnp.float32)
    g = jnp.dot(onehot, gam, preferred_element_type=jnp.float32)
    b = jnp.dot(onehot, bet, preferred_element_type=jnp.float32)
    return xhat * g + b


def _ring_body(t_hbm, x_hbm, g_ref, b_ref, o_hbm,
               x_buf, t_buf, o_buf, in_sem, t_sem, out_sem,
               *, num_types, block, steps):
    gam = g_ref[...]
    bet = b_ref[...]

    def start_in(slot, step):
        pltpu.make_async_copy(
            x_hbm.at[pl.ds(step * block, block)], x_buf.at[slot],
            in_sem.at[slot]).start()
        pltpu.make_async_copy(
            t_hbm.at[pl.ds(step * block, block)], t_buf.at[slot],
            t_sem.at[slot]).start()

    def wait_in(slot):
        pltpu.make_async_copy(
            x_hbm.at[pl.ds(0, block)], x_buf.at[slot],
            in_sem.at[slot]).wait()
        pltpu.make_async_copy(
            t_hbm.at[pl.ds(0, block)], t_buf.at[slot],
            t_sem.at[slot]).wait()

    def start_out(slot, step):
        pltpu.make_async_copy(
            o_buf.at[slot], o_hbm.at[pl.ds(step * block, block)],
            out_sem.at[slot]).start()

    def wait_out(slot):
        pltpu.make_async_copy(
            o_buf.at[slot], o_hbm.at[pl.ds(0, block)],
            out_sem.at[slot]).wait()

    for s in range(_PREFETCH):
        start_in(s % _NBUF, s)

    def body(step, carry):
        slot = jax.lax.rem(step, _NBUF)

        @pl.when(step >= _NBUF)
        def _():
            wait_out(slot)

        @pl.when(step + _PREFETCH < steps)
        def _():
            start_in(jax.lax.rem(step + _PREFETCH, _NBUF), step + _PREFETCH)

        wait_in(slot)
        x = x_buf[slot]
        t = t_buf[slot]
        o_buf[slot] = _compute(x, t, gam, bet, num_types)
        start_out(slot, step)
        return carry

    jax.lax.fori_loop(0, steps, body, 0)

    for s in range(max(steps - _NBUF, 0), steps):
        wait_out(s % _NBUF)


def kernel(type_list, abstract_features, gamma, beta):
    n, d = abstract_features.shape
    num_types = gamma.shape[0]
    t2 = type_list.astype(jnp.int32).reshape(n, 1)

    block = 4000
    if n % block != 0:
        block = 1024
    steps = n // block

    return pl.pallas_call(
        functools.partial(_ring_body, num_types=num_types, block=block,
                          steps=steps),
        out_shape=jax.ShapeDtypeStruct((n, d), jnp.float32),
        in_specs=[
            pl.BlockSpec(memory_space=pltpu.MemorySpace.HBM),
            pl.BlockSpec(memory_space=pltpu.MemorySpace.HBM),
            pl.BlockSpec(memory_space=pltpu.VMEM),
            pl.BlockSpec(memory_space=pltpu.VMEM),
        ],
        out_specs=pl.BlockSpec(memory_space=pltpu.MemorySpace.HBM),
        scratch_shapes=[
            pltpu.VMEM((_NBUF, block, d), jnp.float32),
            pltpu.VMEM((_NBUF, block, 1), jnp.int32),
            pltpu.VMEM((_NBUF, block, d), jnp.float32),
            pltpu.SemaphoreType.DMA((_NBUF,)),
            pltpu.SemaphoreType.DMA((_NBUF,)),
            pltpu.SemaphoreType.DMA((_NBUF,)),
        ],
        name="typenorm_ring",
    )(t2, abstract_features, gamma, beta)


# speedup vs baseline: 1.0959x; 1.0056x over previous
import functools

import jax
import jax.numpy as jnp
from jax.experimental import pallas as pl
from jax.experimental.pallas import tpu as pltpu

_EPS = 1e-5
_NBUF = 4
_PREFETCH = 2


def _compute(x, t, gam, bet, num_types):
    mean = jnp.mean(x, axis=1, keepdims=True)
    xc = x - mean
    var = jnp.mean(xc * xc, axis=1, keepdims=True)
    xhat = xc * jax.lax.rsqrt(var + _EPS)
    onehot = (t == jax.lax.broadcasted_iota(
        jnp.int32, (t.shape[0], num_types), 1)).astype(jnp.float32)
    g = jnp.dot(onehot, gam, preferred_element_type=jnp.float32)
    b = jnp.dot(onehot, bet, preferred_element_type=jnp.float32)
    return xhat * g + b


def _ring_body(t_hbm, x_hbm, g_ref, b_ref, o_hbm,
               x_buf, t_buf, o_buf, in_sem, t_sem, out_sem,
               *, num_types, block, steps):
    gam = g_ref[...]
    bet = b_ref[...]

    def start_in(slot, step):
        pltpu.make_async_copy(
            x_hbm.at[pl.ds(step * block, block)], x_buf.at[slot],
            in_sem.at[slot]).start()
        pltpu.make_async_copy(
            t_hbm.at[pl.ds(step * block, block)], t_buf.at[slot],
            t_sem.at[slot]).start()

    def wait_in(slot):
        pltpu.make_async_copy(
            x_hbm.at[pl.ds(0, block)], x_buf.at[slot],
            in_sem.at[slot]).wait()
        pltpu.make_async_copy(
            t_hbm.at[pl.ds(0, block)], t_buf.at[slot],
            t_sem.at[slot]).wait()

    def start_out(slot, step):
        pltpu.make_async_copy(
            o_buf.at[slot], o_hbm.at[pl.ds(step * block, block)],
            out_sem.at[slot]).start()

    def wait_out(slot):
        pltpu.make_async_copy(
            o_buf.at[slot], o_hbm.at[pl.ds(0, block)],
            out_sem.at[slot]).wait()

    for s in range(_PREFETCH):
        start_in(s % _NBUF, s)

    def body(step, carry):
        slot = jax.lax.rem(step, _NBUF)

        @pl.when(step >= _NBUF)
        def _():
            wait_out(slot)

        @pl.when(step + _PREFETCH < steps)
        def _():
            start_in(jax.lax.rem(step + _PREFETCH, _NBUF), step + _PREFETCH)

        wait_in(slot)
        x = x_buf[slot]
        t = t_buf[slot]
        o_buf[slot] = _compute(x, t, gam, bet, num_types)
        start_out(slot, step)
        return carry

    jax.lax.fori_loop(0, steps, body, 0)

    for s in range(max(steps - _NBUF, 0), steps):
        wait_out(s % _NBUF)


def kernel(type_list, abstract_features, gamma, beta):
    n, d = abstract_features.shape
    num_types = gamma.shape[0]
    t2 = type_list.astype(jnp.int32).reshape(n, 1)

    block = 8000
    if n % block != 0:
        block = 1024
    steps = n // block

    return pl.pallas_call(
        functools.partial(_ring_body, num_types=num_types, block=block,
                          steps=steps),
        out_shape=jax.ShapeDtypeStruct((n, d), jnp.float32),
        in_specs=[
            pl.BlockSpec(memory_space=pltpu.MemorySpace.HBM),
            pl.BlockSpec(memory_space=pltpu.MemorySpace.HBM),
            pl.BlockSpec(memory_space=pltpu.VMEM),
            pl.BlockSpec(memory_space=pltpu.VMEM),
        ],
        out_specs=pl.BlockSpec(memory_space=pltpu.MemorySpace.HBM),
        scratch_shapes=[
            pltpu.VMEM((_NBUF, block, d), jnp.float32),
            pltpu.VMEM((_NBUF, block, 1), jnp.int32),
            pltpu.VMEM((_NBUF, block, d), jnp.float32),
            pltpu.SemaphoreType.DMA((_NBUF,)),
            pltpu.SemaphoreType.DMA((_NBUF,)),
            pltpu.SemaphoreType.DMA((_NBUF,)),
        ],
        name="typenorm_ring",
    )(t2, abstract_features, gamma, beta)
